# baseline (speedup 1.0000x reference)
import jax
import jax.numpy as jnp
from jax.experimental import pallas as pl

_K = 5


def _cheb_kernel(a_ref, y_ref, w1_ref, b1_ref, w2_ref, b2_ref, out_ref):
    A = a_ref[...]
    N = A.shape[0]
    BCN = y_ref.shape[0]
    C = w1_ref.shape[1]
    nb = BCN // C

    d = jnp.sum(A, axis=1)
    inv = jnp.where(d > 0, 1.0 / jnp.sqrt(d), 0.0)
    row = jax.lax.broadcasted_iota(jnp.int32, (N, N), 0)
    col = jax.lax.broadcasted_iota(jnp.int32, (N, N), 1)
    eye = jnp.where(row == col, jnp.float32(1.0), jnp.float32(0.0))
    L = eye - inv[:, None] * A * inv[None, :]

    def proj(T, wt):
        blocks = [jnp.dot(wt, T[b * C:(b + 1) * C, :],
                          preferred_element_type=jnp.float32)
                  for b in range(nb)]
        return jnp.concatenate(blocks, axis=0)

    def layer(Y, w_ref, b_ref):
        acc = proj(Y, w_ref[0])
        T0 = Y
        T1 = jnp.dot(Y, L, preferred_element_type=jnp.float32)
        acc = acc + proj(T1, w_ref[1])
        for k in range(2, _K):
            T2 = 2.0 * jnp.dot(T1, L, preferred_element_type=jnp.float32) - T0
            acc = acc + proj(T2, w_ref[k])
            T0, T1 = T1, T2
        return jnp.maximum(acc + b_ref[...], 0.0)

    h = layer(y_ref[...], w1_ref, b1_ref)
    out_ref[...] = layer(h, w2_ref, b2_ref)


def kernel(graph, flow_x, W1, b1, W2, b2):
    B, N, H, D = flow_x.shape
    C = H * D
    y = flow_x.reshape(B, N, C).transpose(0, 2, 1).reshape(B * C, N)
    b1col = jnp.tile(b1, B).reshape(B * C, 1)
    b2col = jnp.tile(b2, B).reshape(B * C, 1)
    out = pl.pallas_call(
        _cheb_kernel,
        out_shape=jax.ShapeDtypeStruct((B * C, N), jnp.float32),
    )(graph, y, W1.transpose(0, 2, 1), b1col,
      W2.transpose(0, 2, 1), b2col)
    return out.reshape(B, C, N).transpose(0, 2, 1).reshape(B, N, 1, C)


# speedup vs baseline: 1.0764x; 1.0764x over previous
import jax
import jax.numpy as jnp
from jax.experimental import pallas as pl

_K = 5


def _pairblock(W):
    K, C, _ = W.shape
    z = jnp.zeros((K, C, C), W.dtype)
    top = jnp.concatenate([W, z], axis=2)
    bot = jnp.concatenate([z, W], axis=2)
    return jnp.concatenate([top, bot], axis=1)


def _cheb_kernel(a_ref, y_ref, w1_ref, b1_ref, w2_ref, b2_ref, out_ref):
    A = a_ref[...]
    N = A.shape[0]
    BCN = y_ref.shape[0]

    d = jnp.sum(A, axis=1)
    inv = jnp.where(d > 0, 1.0 / jnp.sqrt(d), 0.0)
    row = jax.lax.broadcasted_iota(jnp.int32, (N, N), 0)
    col = jax.lax.broadcasted_iota(jnp.int32, (N, N), 1)
    eye = jnp.where(row == col, jnp.float32(1.0), jnp.float32(0.0))
    L = eye - inv[:, None] * A * inv[None, :]

    P = w1_ref.shape[1]
    npair = BCN // P

    def proj(T, wt):
        blocks = [jnp.dot(wt, T[p * P:(p + 1) * P, :],
                          preferred_element_type=jnp.float32)
                  for p in range(npair)]
        return jnp.concatenate(blocks, axis=0)

    def layer(Y, w_ref, b_ref):
        acc = proj(Y, w_ref[0])
        T0 = Y
        T1 = jnp.dot(Y, L, preferred_element_type=jnp.float32)
        acc = acc + proj(T1, w_ref[1])
        for k in range(2, _K):
            T2 = 2.0 * jnp.dot(T1, L, preferred_element_type=jnp.float32) - T0
            acc = acc + proj(T2, w_ref[k])
            T0, T1 = T1, T2
        return jnp.maximum(acc + b_ref[...], 0.0)

    h = layer(y_ref[...], w1_ref, b1_ref)
    out_ref[...] = layer(h, w2_ref, b2_ref)


def kernel(graph, flow_x, W1, b1, W2, b2):
    B, N, H, D = flow_x.shape
    C = H * D
    y = flow_x.reshape(B, N, C).transpose(0, 2, 1).reshape(B * C, N)
    b1col = jnp.tile(b1, B).reshape(B * C, 1)
    b2col = jnp.tile(b2, B).reshape(B * C, 1)
    out = pl.pallas_call(
        _cheb_kernel,
        out_shape=jax.ShapeDtypeStruct((B * C, N), jnp.float32),
    )(graph, y, _pairblock(W1.transpose(0, 2, 1)), b1col,
      _pairblock(W2.transpose(0, 2, 1)), b2col)
    return out.reshape(B, C, N).transpose(0, 2, 1).reshape(B, N, 1, C)
